# Initial kernel scaffold; baseline (speedup 1.0000x reference)
import jax
import jax.numpy as jnp
from jax import lax
from jax.experimental import pallas as pl
from jax.experimental.pallas import tpu as pltpu
from jax.experimental.pallas import tpu_sc as plsc

E = 1600000
D = 32
C = 400
STREAMS = ((0, 128), (128, 128), (256, 128), (384, 16))

_info = plsc.get_sparse_core_info()
NC, NS = _info.num_cores, _info.num_subcores
NW = NC * NS
PER_W = E // NW
NCHUNK = PER_W // C


def _body(i0_hbm, i1_hbm, i2_hbm, w0_hbm, w1_hbm, w2_hbm, out_hbm,
          i0_v, i1_v, i2_v, r0_v, r1_v, r2_v, sem):
    w = lax.axis_index("s") * NC + lax.axis_index("c")
    w_base = w * PER_W

    def chunk_body(k, carry):
        base = w_base + k * C
        pltpu.sync_copy(i0_hbm.at[pl.ds(base, C)], i0_v)
        pltpu.sync_copy(i1_hbm.at[pl.ds(base, C)], i1_v)
        pltpu.sync_copy(i2_hbm.at[pl.ds(base, C)], i2_v)
        copies = []
        for tbl, iv, rv in ((w0_hbm, i0_v, r0_v),
                            (w1_hbm, i1_v, r1_v),
                            (w2_hbm, i2_v, r2_v)):
            for off, n in STREAMS:
                copies.append(pltpu.async_copy(
                    tbl.at[iv.at[pl.ds(off, n)]], rv.at[pl.ds(off, n)], sem))
        for cp in copies:
            cp.wait()

        def add_row(i, c2):
            for h in range(D // 16):
                sl = pl.ds(h * 16, 16)
                r0_v[i, sl] = r0_v[i, sl] + r1_v[i, sl] + r2_v[i, sl]
            return c2

        lax.fori_loop(0, C, add_row, 0)
        pltpu.sync_copy(r0_v, out_hbm.at[pl.ds(base, C)])
        return carry

    lax.fori_loop(0, NCHUNK, chunk_body, 0)


def kernel(edge_attr, W0, W1, W2):
    idx0 = edge_attr[:, 0]
    idx1 = edge_attr[:, 1]
    idx2 = edge_attr[:, 2]
    run = pl.kernel(
        _body,
        out_type=jax.ShapeDtypeStruct((E, D), jnp.float32),
        mesh=plsc.VectorSubcoreMesh(core_axis_name="c", subcore_axis_name="s"),
        scratch_types=[
            pltpu.VMEM((C,), jnp.int32),
            pltpu.VMEM((C,), jnp.int32),
            pltpu.VMEM((C,), jnp.int32),
            pltpu.VMEM((C, D), jnp.float32),
            pltpu.VMEM((C, D), jnp.float32),
            pltpu.VMEM((C, D), jnp.float32),
            pltpu.SemaphoreType.DMA,
        ],
    )
    return run(idx0, idx1, idx2, W0, W1, W2)


# speedup vs baseline: 6.5130x; 6.5130x over previous
import jax
import jax.numpy as jnp
from jax import lax
from jax.experimental import pallas as pl
from jax.experimental.pallas import tpu as pltpu
from jax.experimental.pallas import tpu_sc as plsc

E = 1600000
D = 32
C = 400
STREAMS = ((0, 128), (128, 128), (256, 128), (384, 16))

_info = plsc.get_sparse_core_info()
NC, NS = _info.num_cores, _info.num_subcores
NW = NC * NS
PER_W = E // NW
NCHUNK = PER_W // C


def _body(i0_hbm, i1_hbm, i2_hbm, w0_hbm, w1_hbm, w2_hbm, out_hbm,
          i0_v, i1_v, i2_v, r0_v, r1_v, r2_v, sem):
    w = lax.axis_index("s") * NC + lax.axis_index("c")
    w_base = w * PER_W

    def chunk_body(k, carry):
        base = w_base + k * C
        pltpu.sync_copy(i0_hbm.at[pl.ds(base, C)], i0_v)
        pltpu.sync_copy(i1_hbm.at[pl.ds(base, C)], i1_v)
        pltpu.sync_copy(i2_hbm.at[pl.ds(base, C)], i2_v)
        copies = []
        for tbl, iv, rv in ((w0_hbm, i0_v, r0_v),
                            (w1_hbm, i1_v, r1_v),
                            (w2_hbm, i2_v, r2_v)):
            for off, n in STREAMS:
                copies.append(pltpu.async_copy(
                    tbl.at[iv.at[pl.ds(off, n)]], rv.at[pl.ds(off, n)], sem))
        for cp in copies:
            cp.wait()

        def add_row(i, c2):
            for h in range(D // 16):
                sl = pl.ds(h * 16, 16)
                r0_v[i, sl] = r0_v[i, sl] + r1_v[i, sl] + r2_v[i, sl]
            return c2

        lax.fori_loop(0, C, add_row, 0)
        pltpu.sync_copy(r0_v, out_hbm.at[pl.ds(base, C)])
        return carry

    lax.fori_loop(0, NCHUNK, chunk_body, 0)


def kernel(edge_attr, W0, W1, W2):
    idx0 = edge_attr[:, 0]
    idx1 = edge_attr[:, 1]
    idx2 = edge_attr[:, 2]
    run = pl.kernel(
        _body,
        out_type=jax.ShapeDtypeStruct((E, D), jnp.float32),
        mesh=plsc.VectorSubcoreMesh(core_axis_name="c", subcore_axis_name="s"),
        compiler_params=pltpu.CompilerParams(use_tc_tiling_on_sc=False),
        scratch_types=[
            pltpu.VMEM((C,), jnp.int32),
            pltpu.VMEM((C,), jnp.int32),
            pltpu.VMEM((C,), jnp.int32),
            pltpu.VMEM((C, D), jnp.float32),
            pltpu.VMEM((C, D), jnp.float32),
            pltpu.VMEM((C, D), jnp.float32),
            pltpu.SemaphoreType.DMA,
        ],
    )
    return run(idx0, idx1, idx2, W0, W1, W2)


# speedup vs baseline: 7.7854x; 1.1954x over previous
import jax
import jax.numpy as jnp
from jax import lax
from jax.experimental import pallas as pl
from jax.experimental.pallas import tpu as pltpu
from jax.experimental.pallas import tpu_sc as plsc

E = 1600000
D = 32
C = 400
STREAMS = ((0, 128), (128, 128), (256, 128), (384, 16))
UNROLL = 8

_info = plsc.get_sparse_core_info()
NC, NS = _info.num_cores, _info.num_subcores
NW = NC * NS
PER_W = E // NW
NCHUNK = PER_W // C
NPAIR = (NCHUNK - 1) // 2


def _body(i0_hbm, i1_hbm, i2_hbm, w0_hbm, w1_hbm, w2_hbm, out_hbm,
          iv, rv, ov, isem, gsem, osem):
    w = lax.axis_index("s") * NC + lax.axis_index("c")
    w_base = w * PER_W
    tables = (w0_hbm, w1_hbm, w2_hbm)
    idx_srcs = (i0_hbm, i1_hbm, i2_hbm)

    def chunk_base(k):
        return w_base + jnp.minimum(k, NCHUNK - 1) * C

    def load_idx(b, k):
        base = chunk_base(k)
        for t in range(3):
            pltpu.async_copy(idx_srcs[t].at[pl.ds(base, C)], iv.at[b, t],
                             isem.at[b])

    def wait_idx(b):
        for t in range(3):
            pltpu.make_async_copy(idx_srcs[t].at[pl.ds(w_base, C)],
                                  iv.at[b, t], isem.at[b]).wait()

    def fire_gathers(b):
        for t in range(3):
            for off, n in STREAMS:
                pltpu.async_copy(
                    tables[t].at[iv.at[b, t].at[pl.ds(off, n)]],
                    rv.at[b, t].at[pl.ds(off, n)], gsem.at[b])

    def wait_gathers(b):
        for t in range(3):
            for off, n in STREAMS:
                pltpu.make_async_copy(
                    tables[t].at[iv.at[b, t].at[pl.ds(off, n)]],
                    rv.at[b, t].at[pl.ds(off, n)], gsem.at[b]).wait()

    def compute_store(b, k, first):
        @pl.when(jnp.logical_not(first))
        def _():
            pltpu.make_async_copy(ov.at[b], out_hbm.at[pl.ds(w_base, C)],
                                  osem.at[b]).wait()

        @plsc.parallel_loop(0, C, 1, unroll=UNROLL)
        def _(i):
            for h in range(D // 16):
                sl = pl.ds(h * 16, 16)
                ov[b, i, sl] = (rv[b, 0, i, sl] + rv[b, 1, i, sl]
                                + rv[b, 2, i, sl])

        pltpu.async_copy(ov.at[b], out_hbm.at[pl.ds(chunk_base(k), C)],
                         osem.at[b])

    load_idx(0, 0)
    wait_idx(0)
    fire_gathers(0)
    load_idx(1, 1)

    def pair_body(p, carry):
        k = p * 2
        wait_idx(1)
        fire_gathers(1)
        wait_gathers(0)
        compute_store(0, k, p == 0)
        load_idx(0, k + 2)
        wait_idx(0)
        fire_gathers(0)
        wait_gathers(1)
        compute_store(1, k + 1, p == 0)
        load_idx(1, k + 3)
        return carry

    lax.fori_loop(0, NPAIR, pair_body, 0)

    wait_gathers(0)
    compute_store(0, NCHUNK - 1, False)
    wait_idx(1)
    for b in range(2):
        pltpu.make_async_copy(ov.at[b], out_hbm.at[pl.ds(w_base, C)],
                              osem.at[b]).wait()


def kernel(edge_attr, W0, W1, W2):
    idx0 = edge_attr[:, 0]
    idx1 = edge_attr[:, 1]
    idx2 = edge_attr[:, 2]
    run = pl.kernel(
        _body,
        out_type=jax.ShapeDtypeStruct((E, D), jnp.float32),
        mesh=plsc.VectorSubcoreMesh(core_axis_name="c", subcore_axis_name="s"),
        compiler_params=pltpu.CompilerParams(use_tc_tiling_on_sc=False),
        scratch_types=[
            pltpu.VMEM((2, 3, C), jnp.int32),
            pltpu.VMEM((2, 3, C, D), jnp.float32),
            pltpu.VMEM((2, C, D), jnp.float32),
            pltpu.SemaphoreType.DMA((2,)),
            pltpu.SemaphoreType.DMA((2,)),
            pltpu.SemaphoreType.DMA((2,)),
        ],
    )
    return run(idx0, idx1, idx2, W0, W1, W2)
